# Initial kernel scaffold; baseline (speedup 1.0000x reference)
import jax
import jax.numpy as jnp
from jax.experimental import pallas as pl


def kernel(input, target):
    raise NotImplementedError("write your pallas kernel here")


# speedup vs baseline: 46.4916x; 46.4916x over previous
import functools

import jax
import jax.numpy as jnp
from jax import lax
from jax.experimental import pallas as pl
from jax.experimental.pallas import tpu as pltpu
from jax.experimental.pallas import tpu_sc as plsc

TRIM_FRAC = 0.1
N = 4 * 4096 * 1024
TRIM = int(N * TRIM_FRAC)
KEPT = N - 2 * TRIM

NB = 32768
SHIFT = 17
NW = 32
PW = N // NW
CH = 8192
NCH = PW // CH
LANES = 16

RB, CB = NB // 128, 128


def _hist_body(inp, tgt, cnt_out, sum_out, in_buf, tg_buf, cnt_h, sum_h):
    wid = lax.axis_index("s") * 2 + lax.axis_index("c")
    base = wid * PW

    def zero(i, _):
        cnt_h[pl.ds(i * LANES, LANES)] = jnp.zeros((LANES,), jnp.int32)
        sum_h[pl.ds(i * LANES, LANES)] = jnp.zeros((LANES,), jnp.float32)
        return 0

    lax.fori_loop(0, NB // LANES, zero, 0)

    ones = jnp.ones((LANES,), jnp.int32)
    shift = jnp.full((LANES,), SHIFT, jnp.int32)

    def chunk(g, _):
        off = base + g * CH
        pltpu.sync_copy(inp.at[pl.ds(off, CH)], in_buf)
        pltpu.sync_copy(tgt.at[pl.ds(off, CH)], tg_buf)

        def inner(i, _):
            a = in_buf[pl.ds(i * LANES, LANES)]
            b = tg_buf[pl.ds(i * LANES, LANES)]
            d = a - b
            e = d * d
            bits = lax.bitcast_convert_type(e, jnp.int32)
            idx = lax.shift_right_logical(bits, shift)
            plsc.addupdate_scatter(sum_h, [idx], e)
            plsc.addupdate_scatter(cnt_h, [idx], ones)
            return 0

        lax.fori_loop(0, CH // LANES, inner, 0)
        return 0

    lax.fori_loop(0, NCH, chunk, 0)

    pltpu.sync_copy(cnt_h, cnt_out.at[wid])
    pltpu.sync_copy(sum_h, sum_out.at[wid])


_hist = pl.kernel(
    _hist_body,
    out_type=[
        jax.ShapeDtypeStruct((NW, NB), jnp.int32),
        jax.ShapeDtypeStruct((NW, NB), jnp.float32),
    ],
    mesh=plsc.VectorSubcoreMesh(core_axis_name="c", subcore_axis_name="s"),
    compiler_params=pltpu.CompilerParams(needs_layout_passes=False),
    scratch_types=[
        pltpu.VMEM((CH,), jnp.float32),
        pltpu.VMEM((CH,), jnp.float32),
        pltpu.VMEM((NB,), jnp.int32),
        pltpu.VMEM((NB,), jnp.float32),
    ],
)


def _select_body(cnt_ref, sum_ref, out_ref):
    cnt = jnp.sum(cnt_ref[...].astype(jnp.float32), axis=0)
    sums = jnp.sum(sum_ref[...], axis=0)

    ii = lax.broadcasted_iota(jnp.int32, (CB, CB), 0)
    jj = lax.broadcasted_iota(jnp.int32, (CB, CB), 1)
    upper_inc = (ii <= jj).astype(jnp.float32)
    rowcum = jnp.dot(cnt, upper_inc, preferred_element_type=jnp.float32,
                     precision=lax.Precision.HIGHEST)

    ri = lax.broadcasted_iota(jnp.int32, (RB, RB), 0)
    rj = lax.broadcasted_iota(jnp.int32, (RB, RB), 1)
    lower_strict = (rj < ri).astype(jnp.float32)
    rowtot = jnp.sum(cnt, axis=1, keepdims=True)
    prevrows = jnp.dot(lower_strict, rowtot,
                       preferred_element_type=jnp.float32,
                       precision=lax.Precision.HIGHEST)

    cum = rowcum + prevrows
    cum_prev = cum - cnt

    k1 = jnp.float32(TRIM)
    k2 = jnp.float32(N - TRIM)
    zero = jnp.zeros_like(sums)

    sum_below = jnp.sum(jnp.where(cum <= k1, sums, zero))
    lo_bnd = jnp.logical_and(cum_prev < k1, cum > k1)
    lo_cnt = jnp.sum(jnp.where(lo_bnd, cnt, zero))
    lo_sum = jnp.sum(jnp.where(lo_bnd, sums, zero))
    lo_prev = jnp.sum(jnp.where(lo_bnd, cum_prev, zero))
    mean_lo = lo_sum / jnp.maximum(lo_cnt, 1.0)
    bottom = sum_below + (k1 - lo_prev) * mean_lo

    sum_above = jnp.sum(jnp.where(cum_prev >= k2, sums, zero))
    hi_bnd = jnp.logical_and(cum_prev < k2, cum > k2)
    hi_cnt = jnp.sum(jnp.where(hi_bnd, cnt, zero))
    hi_sum = jnp.sum(jnp.where(hi_bnd, sums, zero))
    hi_cum = jnp.sum(jnp.where(hi_bnd, cum, zero))
    mean_hi = hi_sum / jnp.maximum(hi_cnt, 1.0)
    top = sum_above + (hi_cum - k2) * mean_hi

    total = jnp.sum(sums)
    out_ref[0, 0] = (total - bottom - top) / jnp.float32(KEPT)


_select = pl.pallas_call(
    _select_body,
    out_shape=jax.ShapeDtypeStruct((1, 1), jnp.float32),
    out_specs=pl.BlockSpec(memory_space=pltpu.SMEM),
)


@jax.jit
def kernel(input, target):
    inp = input.reshape(N)
    tgt = target.reshape(N)
    cnt, sums = _hist(inp, tgt)
    res = _select(cnt.reshape(NW, RB, CB), sums.reshape(NW, RB, CB))
    return res[0, 0]


# speedup vs baseline: 118.7253x; 2.5537x over previous
import functools

import jax
import jax.numpy as jnp
from jax import lax
from jax.experimental import pallas as pl
from jax.experimental.pallas import tpu as pltpu
from jax.experimental.pallas import tpu_sc as plsc

TRIM_FRAC = 0.1
N = 4 * 4096 * 1024
TRIM = int(N * TRIM_FRAC)
KEPT = N - 2 * TRIM

NB = 32768
SHIFT = 17
NW = 32
PW = N // NW
CH = 8192
NCH = PW // CH
LANES = 16

RB, CB = NB // 128, 128


UNROLL = 8


def _hist_body(inp, tgt, cnt_out, sum_out,
               in_bufs, tg_bufs, cnt_h, sum_h, in_sems, tg_sems):
    wid = lax.axis_index("s") * 2 + lax.axis_index("c")
    base = wid * PW

    @plsc.parallel_loop(0, NB // LANES, 1, unroll=UNROLL)
    def _zero(i):
        o = i * LANES
        cnt_h[pl.ds(o, LANES)] = jnp.zeros((LANES,), jnp.int32)
        sum_h[pl.ds(o, LANES)] = jnp.zeros((LANES,), jnp.float32)

    ones = jnp.ones((LANES,), jnp.int32)
    shift = jnp.full((LANES,), SHIFT, jnp.int32)

    def copies(g, b):
        off = base + g * CH
        return (
            pltpu.make_async_copy(inp.at[pl.ds(off, CH)], in_bufs[b],
                                  in_sems.at[b]),
            pltpu.make_async_copy(tgt.at[pl.ds(off, CH)], tg_bufs[b],
                                  tg_sems.at[b]),
        )

    for b in range(2):
        for c in copies(b, b):
            c.start()

    def chunk_pair(j, _):
        for b in range(2):
            g = 2 * j + b
            for c in copies(g, b):
                c.wait()

            @plsc.parallel_loop(0, CH // LANES, 1, unroll=UNROLL)
            def _inner(i, b=b):
                o = i * LANES
                a = in_bufs[b][pl.ds(o, LANES)]
                t = tg_bufs[b][pl.ds(o, LANES)]
                d = a - t
                e = d * d
                bits = lax.bitcast_convert_type(e, jnp.int32)
                idx = lax.shift_right_logical(bits, shift)
                plsc.addupdate_scatter(sum_h, [idx], e)
                plsc.addupdate_scatter(cnt_h, [idx], ones)

            @pl.when(g + 2 < NCH)
            def _(g=g, b=b):
                for c in copies(g + 2, b):
                    c.start()
        return 0

    lax.fori_loop(0, NCH // 2, chunk_pair, 0)

    pltpu.sync_copy(cnt_h, cnt_out.at[wid])
    pltpu.sync_copy(sum_h, sum_out.at[wid])


_hist = pl.kernel(
    _hist_body,
    out_type=[
        jax.ShapeDtypeStruct((NW, NB), jnp.int32),
        jax.ShapeDtypeStruct((NW, NB), jnp.float32),
    ],
    mesh=plsc.VectorSubcoreMesh(core_axis_name="c", subcore_axis_name="s"),
    compiler_params=pltpu.CompilerParams(needs_layout_passes=False),
    scratch_types=[
        [pltpu.VMEM((CH,), jnp.float32)] * 2,
        [pltpu.VMEM((CH,), jnp.float32)] * 2,
        pltpu.VMEM((NB,), jnp.int32),
        pltpu.VMEM((NB,), jnp.float32),
        pltpu.SemaphoreType.DMA((2,)),
        pltpu.SemaphoreType.DMA((2,)),
    ],
)


def _select_body(cnt_ref, sum_ref, out_ref):
    cnt = jnp.sum(cnt_ref[...].astype(jnp.float32), axis=0)
    sums = jnp.sum(sum_ref[...], axis=0)

    ii = lax.broadcasted_iota(jnp.int32, (CB, CB), 0)
    jj = lax.broadcasted_iota(jnp.int32, (CB, CB), 1)
    upper_inc = (ii <= jj).astype(jnp.float32)
    rowcum = jnp.dot(cnt, upper_inc, preferred_element_type=jnp.float32,
                     precision=lax.Precision.HIGHEST)

    ri = lax.broadcasted_iota(jnp.int32, (RB, RB), 0)
    rj = lax.broadcasted_iota(jnp.int32, (RB, RB), 1)
    lower_strict = (rj < ri).astype(jnp.float32)
    rowtot = jnp.sum(cnt, axis=1, keepdims=True)
    prevrows = jnp.dot(lower_strict, rowtot,
                       preferred_element_type=jnp.float32,
                       precision=lax.Precision.HIGHEST)

    cum = rowcum + prevrows
    cum_prev = cum - cnt

    k1 = jnp.float32(TRIM)
    k2 = jnp.float32(N - TRIM)
    zero = jnp.zeros_like(sums)

    sum_below = jnp.sum(jnp.where(cum <= k1, sums, zero))
    lo_bnd = jnp.logical_and(cum_prev < k1, cum > k1)
    lo_cnt = jnp.sum(jnp.where(lo_bnd, cnt, zero))
    lo_sum = jnp.sum(jnp.where(lo_bnd, sums, zero))
    lo_prev = jnp.sum(jnp.where(lo_bnd, cum_prev, zero))
    mean_lo = lo_sum / jnp.maximum(lo_cnt, 1.0)
    bottom = sum_below + (k1 - lo_prev) * mean_lo

    sum_above = jnp.sum(jnp.where(cum_prev >= k2, sums, zero))
    hi_bnd = jnp.logical_and(cum_prev < k2, cum > k2)
    hi_cnt = jnp.sum(jnp.where(hi_bnd, cnt, zero))
    hi_sum = jnp.sum(jnp.where(hi_bnd, sums, zero))
    hi_cum = jnp.sum(jnp.where(hi_bnd, cum, zero))
    mean_hi = hi_sum / jnp.maximum(hi_cnt, 1.0)
    top = sum_above + (hi_cum - k2) * mean_hi

    total = jnp.sum(sums)
    out_ref[0, 0] = (total - bottom - top) / jnp.float32(KEPT)


_select = pl.pallas_call(
    _select_body,
    out_shape=jax.ShapeDtypeStruct((1, 1), jnp.float32),
    out_specs=pl.BlockSpec(memory_space=pltpu.SMEM),
)


@jax.jit
def kernel(input, target):
    inp = input.reshape(N)
    tgt = target.reshape(N)
    cnt, sums = _hist(inp, tgt)
    res = _select(cnt.reshape(NW, RB, CB), sums.reshape(NW, RB, CB))
    return res[0, 0]


# speedup vs baseline: 210.7601x; 1.7752x over previous
import functools

import jax
import jax.numpy as jnp
from jax import lax
from jax.experimental import pallas as pl
from jax.experimental.pallas import tpu as pltpu
from jax.experimental.pallas import tpu_sc as plsc

TRIM_FRAC = 0.1
N = 4 * 4096 * 1024
TRIM = int(N * TRIM_FRAC)
KEPT = N - 2 * TRIM

NB = 32768
SHIFT = 17
NW = 32
LANES = 16

NCOL = 1024
NROW = N // NCOL
RPW = NROW // NW
CROWS = 8
NCH = RPW // CROWS
UNROLL = 8

RB, CB = NB // 128, 128


def _hist_body(inp, tgt, cnt_out, sum_out,
               in_bufs, tg_bufs, cnt_h, sum_h, in_sems, tg_sems):
    wid = lax.axis_index("s") * 2 + lax.axis_index("c")
    base = wid * RPW

    @plsc.parallel_loop(0, NB // LANES, 1, unroll=UNROLL)
    def _zero(i):
        o = i * LANES
        cnt_h[pl.ds(o, LANES)] = jnp.zeros((LANES,), jnp.int32)
        sum_h[pl.ds(o, LANES)] = jnp.zeros((LANES,), jnp.float32)

    ones = jnp.ones((LANES,), jnp.int32)
    shift = jnp.full((LANES,), SHIFT, jnp.int32)

    def copies(g, b):
        row0 = base + g * CROWS
        return (
            pltpu.make_async_copy(inp.at[pl.ds(row0, CROWS)], in_bufs[b],
                                  in_sems.at[b]),
            pltpu.make_async_copy(tgt.at[pl.ds(row0, CROWS)], tg_bufs[b],
                                  tg_sems.at[b]),
        )

    for b in range(2):
        for c in copies(b, b):
            c.start()

    def chunk_pair(j, _):
        for b in range(2):
            g = 2 * j + b
            for c in copies(g, b):
                c.wait()

            for r in range(CROWS):
                @plsc.parallel_loop(0, NCOL // LANES, 1, unroll=UNROLL)
                def _inner(i, b=b, r=r):
                    o = i * LANES
                    a = in_bufs[b][r, pl.ds(o, LANES)]
                    t = tg_bufs[b][r, pl.ds(o, LANES)]
                    d = a - t
                    e = d * d
                    bits = lax.bitcast_convert_type(e, jnp.int32)
                    idx = lax.shift_right_logical(bits, shift)
                    plsc.addupdate_scatter(sum_h, [idx], e)
                    plsc.addupdate_scatter(cnt_h, [idx], ones)

            @pl.when(g + 2 < NCH)
            def _(g=g, b=b):
                for c in copies(g + 2, b):
                    c.start()
        return 0

    lax.fori_loop(0, NCH // 2, chunk_pair, 0)

    pltpu.sync_copy(cnt_h, cnt_out.at[pl.ds(wid * NB, NB)])
    pltpu.sync_copy(sum_h, sum_out.at[pl.ds(wid * NB, NB)])


_hist = pl.kernel(
    _hist_body,
    out_type=[
        jax.ShapeDtypeStruct((NW * NB,), jnp.int32),
        jax.ShapeDtypeStruct((NW * NB,), jnp.float32),
    ],
    mesh=plsc.VectorSubcoreMesh(core_axis_name="c", subcore_axis_name="s"),
    compiler_params=pltpu.CompilerParams(
        needs_layout_passes=False,
        use_tc_tiling_on_sc=True,
    ),
    scratch_types=[
        [pltpu.VMEM((CROWS, NCOL), jnp.float32)] * 2,
        [pltpu.VMEM((CROWS, NCOL), jnp.float32)] * 2,
        pltpu.VMEM((NB,), jnp.int32),
        pltpu.VMEM((NB,), jnp.float32),
        pltpu.SemaphoreType.DMA((2,)),
        pltpu.SemaphoreType.DMA((2,)),
    ],
)


def _select_body(cnt_ref, sum_ref, out_ref):
    cnt = jnp.sum(cnt_ref[...].astype(jnp.float32), axis=0)
    sums = jnp.sum(sum_ref[...], axis=0)

    ii = lax.broadcasted_iota(jnp.int32, (CB, CB), 0)
    jj = lax.broadcasted_iota(jnp.int32, (CB, CB), 1)
    upper_inc = (ii <= jj).astype(jnp.float32)
    rowcum = jnp.dot(cnt, upper_inc, preferred_element_type=jnp.float32,
                     precision=lax.Precision.HIGHEST)

    ri = lax.broadcasted_iota(jnp.int32, (RB, RB), 0)
    rj = lax.broadcasted_iota(jnp.int32, (RB, RB), 1)
    lower_strict = (rj < ri).astype(jnp.float32)
    rowtot = jnp.sum(cnt, axis=1, keepdims=True)
    prevrows = jnp.dot(lower_strict, rowtot,
                       preferred_element_type=jnp.float32,
                       precision=lax.Precision.HIGHEST)

    cum = rowcum + prevrows
    cum_prev = cum - cnt

    k1 = jnp.float32(TRIM)
    k2 = jnp.float32(N - TRIM)
    zero = jnp.zeros_like(sums)

    sum_below = jnp.sum(jnp.where(cum <= k1, sums, zero))
    lo_bnd = jnp.logical_and(cum_prev < k1, cum > k1)
    lo_cnt = jnp.sum(jnp.where(lo_bnd, cnt, zero))
    lo_sum = jnp.sum(jnp.where(lo_bnd, sums, zero))
    lo_prev = jnp.sum(jnp.where(lo_bnd, cum_prev, zero))
    mean_lo = lo_sum / jnp.maximum(lo_cnt, 1.0)
    bottom = sum_below + (k1 - lo_prev) * mean_lo

    sum_above = jnp.sum(jnp.where(cum_prev >= k2, sums, zero))
    hi_bnd = jnp.logical_and(cum_prev < k2, cum > k2)
    hi_cnt = jnp.sum(jnp.where(hi_bnd, cnt, zero))
    hi_sum = jnp.sum(jnp.where(hi_bnd, sums, zero))
    hi_cum = jnp.sum(jnp.where(hi_bnd, cum, zero))
    mean_hi = hi_sum / jnp.maximum(hi_cnt, 1.0)
    top = sum_above + (hi_cum - k2) * mean_hi

    total = jnp.sum(sums)
    out_ref[0, 0] = (total - bottom - top) / jnp.float32(KEPT)


_select = pl.pallas_call(
    _select_body,
    out_shape=jax.ShapeDtypeStruct((1, 1), jnp.float32),
    out_specs=pl.BlockSpec(memory_space=pltpu.SMEM),
)


@jax.jit
def kernel(input, target):
    inp = input.reshape(NROW, NCOL)
    tgt = target.reshape(NROW, NCOL)
    cnt, sums = _hist(inp, tgt)
    res = _select(cnt.reshape(NW, RB, CB), sums.reshape(NW, RB, CB))
    return res[0, 0]


# speedup vs baseline: 314.7854x; 1.4936x over previous
import functools

import jax
import jax.numpy as jnp
from jax import lax
from jax.experimental import pallas as pl
from jax.experimental.pallas import tpu as pltpu
from jax.experimental.pallas import tpu_sc as plsc

TRIM_FRAC = 0.1
N = 4 * 4096 * 1024
TRIM = int(N * TRIM_FRAC)
KEPT = N - 2 * TRIM

NB = 16384
SHIFT = 17
NW = 32
LANES = 16

NCOL = 1024
NROW = N // NCOL
RPW = NROW // NW
CROWS = 16
NCH = RPW // CROWS
UNROLL = 8

RB, CB = NB // 128, 128


def _hist_body(inp, tgt, cnt_out, tot_out,
               in_bufs, tg_bufs, cnt_h, acc_v, in_sems, tg_sems):
    wid = lax.axis_index("s") * 2 + lax.axis_index("c")
    base = wid * RPW

    @plsc.parallel_loop(0, NB // LANES, 1, unroll=UNROLL)
    def _zero(i):
        cnt_h[pl.ds(i * LANES, LANES)] = jnp.zeros((LANES,), jnp.int32)

    ones = jnp.ones((LANES,), jnp.int32)
    shift = jnp.full((LANES,), SHIFT, jnp.int32)

    def copies(g, b):
        row0 = base + g * CROWS
        return (
            pltpu.make_async_copy(inp.at[pl.ds(row0, CROWS)], in_bufs[b],
                                  in_sems.at[b]),
            pltpu.make_async_copy(tgt.at[pl.ds(row0, CROWS)], tg_bufs[b],
                                  tg_sems.at[b]),
        )

    for b in range(2):
        for c in copies(b, b):
            c.start()

    def chunk_pair(j, acc):
        for b in range(2):
            g = 2 * j + b
            for c in copies(g, b):
                c.wait()

            @plsc.parallel_loop(0, CROWS * NCOL // LANES, 1, unroll=UNROLL,
                                carry=acc)
            def _inner(i, a_sum, b=b):
                r = lax.shift_right_logical(i, 6)
                o = (i & (NCOL // LANES - 1)) * LANES
                a = in_bufs[b][r, pl.ds(o, LANES)]
                t = tg_bufs[b][r, pl.ds(o, LANES)]
                d = a - t
                e = d * d
                bits = lax.bitcast_convert_type(e, jnp.int32)
                idx = lax.shift_right_logical(bits, shift)
                plsc.addupdate_scatter(cnt_h, [idx], ones)
                return a_sum + e

            acc = _inner

            @pl.when(g + 2 < NCH)
            def _(g=g, b=b):
                for c in copies(g + 2, b):
                    c.start()
        return acc

    acc = lax.fori_loop(0, NCH // 2, chunk_pair,
                        jnp.zeros((LANES,), jnp.float32))

    acc_v[...] = acc
    pltpu.sync_copy(cnt_h, cnt_out.at[pl.ds(wid * NB, NB)])
    pltpu.sync_copy(acc_v, tot_out.at[pl.ds(wid * LANES, LANES)])


_hist = pl.kernel(
    _hist_body,
    out_type=[
        jax.ShapeDtypeStruct((NW * NB,), jnp.int32),
        jax.ShapeDtypeStruct((NW * LANES,), jnp.float32),
    ],
    mesh=plsc.VectorSubcoreMesh(core_axis_name="c", subcore_axis_name="s"),
    compiler_params=pltpu.CompilerParams(
        needs_layout_passes=False,
        use_tc_tiling_on_sc=True,
    ),
    scratch_types=[
        [pltpu.VMEM((CROWS, NCOL), jnp.float32)] * 2,
        [pltpu.VMEM((CROWS, NCOL), jnp.float32)] * 2,
        pltpu.VMEM((NB,), jnp.int32),
        pltpu.VMEM((LANES,), jnp.float32),
        pltpu.SemaphoreType.DMA((2,)),
        pltpu.SemaphoreType.DMA((2,)),
    ],
)


def _select_body(cnt_ref, tot_ref, out_ref):
    cnt = jnp.sum(cnt_ref[...].astype(jnp.float32), axis=0)

    rr = lax.broadcasted_iota(jnp.int32, (RB, CB), 0)
    cc = lax.broadcasted_iota(jnp.int32, (RB, CB), 1)
    pat = (rr * CB + cc) * (2 ** SHIFT) + 2 ** (SHIFT - 1)
    mid = lax.bitcast_convert_type(pat, jnp.float32)
    mid = jnp.where(jnp.isfinite(mid), mid, 0.0)
    csum = cnt * mid

    ii = lax.broadcasted_iota(jnp.int32, (CB, CB), 0)
    jj = lax.broadcasted_iota(jnp.int32, (CB, CB), 1)
    upper_inc = (ii <= jj).astype(jnp.float32)
    rowcum = jnp.dot(cnt, upper_inc, preferred_element_type=jnp.float32,
                     precision=lax.Precision.HIGHEST)

    ri = lax.broadcasted_iota(jnp.int32, (RB, RB), 0)
    rj = lax.broadcasted_iota(jnp.int32, (RB, RB), 1)
    lower_strict = (rj < ri).astype(jnp.float32)
    rowtot = jnp.sum(cnt, axis=1, keepdims=True)
    prevrows = jnp.dot(lower_strict, rowtot,
                       preferred_element_type=jnp.float32,
                       precision=lax.Precision.HIGHEST)

    cum = rowcum + prevrows
    cum_prev = cum - cnt

    k1 = jnp.float32(TRIM)
    k2 = jnp.float32(N - TRIM)
    zero = jnp.zeros_like(csum)

    sum_below = jnp.sum(jnp.where(cum <= k1, csum, zero))
    lo_bnd = jnp.logical_and(cum_prev < k1, cum > k1)
    lo_prev = jnp.sum(jnp.where(lo_bnd, cum_prev, zero))
    mid_lo = jnp.sum(jnp.where(lo_bnd, mid, zero))
    bottom = sum_below + (k1 - lo_prev) * mid_lo

    sum_above = jnp.sum(jnp.where(cum_prev >= k2, csum, zero))
    hi_bnd = jnp.logical_and(cum_prev < k2, cum > k2)
    hi_cum = jnp.sum(jnp.where(hi_bnd, cum, zero))
    mid_hi = jnp.sum(jnp.where(hi_bnd, mid, zero))
    top = sum_above + (hi_cum - k2) * mid_hi

    total = jnp.sum(tot_ref[...])
    out_ref[0, 0] = (total - bottom - top) / jnp.float32(KEPT)


_select = pl.pallas_call(
    _select_body,
    out_shape=jax.ShapeDtypeStruct((1, 1), jnp.float32),
    out_specs=pl.BlockSpec(memory_space=pltpu.SMEM),
)


@jax.jit
def kernel(input, target):
    inp = input.reshape(NROW, NCOL)
    tgt = target.reshape(NROW, NCOL)
    cnt, tot = _hist(inp, tgt)
    res = _select(cnt.reshape(NW, RB, CB), tot.reshape(4, 128))
    return res[0, 0]


# speedup vs baseline: 315.1131x; 1.0010x over previous
import functools

import jax
import jax.numpy as jnp
from jax import lax
from jax.experimental import pallas as pl
from jax.experimental.pallas import tpu as pltpu
from jax.experimental.pallas import tpu_sc as plsc

TRIM_FRAC = 0.1
N = 4 * 4096 * 1024
TRIM = int(N * TRIM_FRAC)
KEPT = N - 2 * TRIM

NB = 16384
SHIFT = 17
NW = 32
LANES = 16

NCOL = 1024
NROW = N // NCOL
RPW = NROW // NW
CROWS = 16
NCH = RPW // CROWS
UNROLL = 8

RB, CB = NB // 128, 128


def _hist_body(inp, tgt, cnt_out,
               in_bufs, tg_bufs, cnt_h, in_sems, tg_sems):
    wid = lax.axis_index("s") * 2 + lax.axis_index("c")
    base = wid * RPW

    @plsc.parallel_loop(0, NB // LANES, 1, unroll=UNROLL)
    def _zero(i):
        r = lax.shift_right_logical(i, 3)
        o = (i & 7) * LANES
        cnt_h[r, pl.ds(o, LANES)] = jnp.zeros((LANES,), jnp.int32)

    ones = jnp.ones((LANES,), jnp.int32)
    shift = jnp.full((LANES,), SHIFT, jnp.int32)
    shift_hi = jnp.full((LANES,), SHIFT + 7, jnp.int32)
    colmask = jnp.full((LANES,), CB - 1, jnp.int32)

    def copies(g, b):
        row0 = base + g * CROWS
        return (
            pltpu.make_async_copy(inp.at[pl.ds(row0, CROWS)], in_bufs[b],
                                  in_sems.at[b]),
            pltpu.make_async_copy(tgt.at[pl.ds(row0, CROWS)], tg_bufs[b],
                                  tg_sems.at[b]),
        )

    for b in range(2):
        for c in copies(b, b):
            c.start()

    def chunk_pair(j, acc):
        for b in range(2):
            g = 2 * j + b
            for c in copies(g, b):
                c.wait()

            @plsc.parallel_loop(0, CROWS * NCOL // LANES, 1, unroll=UNROLL,
                                carry=acc)
            def _inner(i, a_sum, b=b):
                r = lax.shift_right_logical(i, 6)
                o = (i & (NCOL // LANES - 1)) * LANES
                a = in_bufs[b][r, pl.ds(o, LANES)]
                t = tg_bufs[b][r, pl.ds(o, LANES)]
                d = a - t
                e = d * d
                bits = lax.bitcast_convert_type(e, jnp.int32)
                brow = lax.shift_right_logical(bits, shift_hi)
                bcol = lax.shift_right_logical(bits, shift) & colmask
                plsc.addupdate_scatter(cnt_h, [brow, bcol], ones)
                return a_sum + e

            acc = _inner

            @pl.when(g + 2 < NCH)
            def _(g=g, b=b):
                for c in copies(g + 2, b):
                    c.start()
        return acc

    acc = lax.fori_loop(0, NCH // 2, chunk_pair,
                        jnp.zeros((LANES,), jnp.float32))

    cnt_h[RB - 1, pl.ds(CB - LANES, LANES)] = (
        lax.bitcast_convert_type(acc, jnp.int32))
    pltpu.sync_copy(cnt_h, cnt_out.at[wid])


_hist = pl.kernel(
    _hist_body,
    out_type=jax.ShapeDtypeStruct((NW, RB, CB), jnp.int32),
    mesh=plsc.VectorSubcoreMesh(core_axis_name="c", subcore_axis_name="s"),
    compiler_params=pltpu.CompilerParams(
        needs_layout_passes=False,
        use_tc_tiling_on_sc=True,
    ),
    scratch_types=[
        [pltpu.VMEM((CROWS, NCOL), jnp.float32)] * 2,
        [pltpu.VMEM((CROWS, NCOL), jnp.float32)] * 2,
        pltpu.VMEM((RB, CB), jnp.int32),
        pltpu.SemaphoreType.DMA((2,)),
        pltpu.SemaphoreType.DMA((2,)),
    ],
)


def _select_body(cnt_ref, out_ref):
    raw = cnt_ref[...]

    rr = lax.broadcasted_iota(jnp.int32, (RB, CB), 0)
    cc = lax.broadcasted_iota(jnp.int32, (RB, CB), 1)
    stash_mask = jnp.logical_and(rr == RB - 1, cc >= CB - LANES)

    total = jnp.sum(jnp.where(stash_mask[None],
                              lax.bitcast_convert_type(raw, jnp.float32),
                              0.0))

    cnt = jnp.sum(raw.astype(jnp.float32), axis=0)
    cnt = jnp.where(stash_mask, 0.0, cnt)

    pat = (rr * CB + cc) * (2 ** SHIFT) + 2 ** (SHIFT - 1)
    mid = lax.bitcast_convert_type(pat, jnp.float32)
    mid = jnp.where(jnp.isfinite(mid), mid, 0.0)
    csum = cnt * mid

    ii = lax.broadcasted_iota(jnp.int32, (CB, CB), 0)
    jj = lax.broadcasted_iota(jnp.int32, (CB, CB), 1)
    upper_inc = (ii <= jj).astype(jnp.float32)
    rowcum = jnp.dot(cnt, upper_inc, preferred_element_type=jnp.float32,
                     precision=lax.Precision.HIGHEST)

    ri = lax.broadcasted_iota(jnp.int32, (RB, RB), 0)
    rj = lax.broadcasted_iota(jnp.int32, (RB, RB), 1)
    lower_strict = (rj < ri).astype(jnp.float32)
    rowtot = jnp.sum(cnt, axis=1, keepdims=True)
    prevrows = jnp.dot(lower_strict, rowtot,
                       preferred_element_type=jnp.float32,
                       precision=lax.Precision.HIGHEST)

    cum = rowcum + prevrows
    cum_prev = cum - cnt

    k1 = jnp.float32(TRIM)
    k2 = jnp.float32(N - TRIM)
    zero = jnp.zeros_like(csum)

    sum_below = jnp.sum(jnp.where(cum <= k1, csum, zero))
    lo_bnd = jnp.logical_and(cum_prev < k1, cum > k1)
    lo_prev = jnp.sum(jnp.where(lo_bnd, cum_prev, zero))
    mid_lo = jnp.sum(jnp.where(lo_bnd, mid, zero))
    bottom = sum_below + (k1 - lo_prev) * mid_lo

    sum_above = jnp.sum(jnp.where(cum_prev >= k2, csum, zero))
    hi_bnd = jnp.logical_and(cum_prev < k2, cum > k2)
    hi_cum = jnp.sum(jnp.where(hi_bnd, cum, zero))
    mid_hi = jnp.sum(jnp.where(hi_bnd, mid, zero))
    top = sum_above + (hi_cum - k2) * mid_hi

    out_ref[0, 0] = (total - bottom - top) / jnp.float32(KEPT)


_select = pl.pallas_call(
    _select_body,
    out_shape=jax.ShapeDtypeStruct((1, 1), jnp.float32),
    out_specs=pl.BlockSpec(memory_space=pltpu.SMEM),
)


@jax.jit
def kernel(input, target):
    inp = input.reshape(NROW, NCOL)
    tgt = target.reshape(NROW, NCOL)
    cnt = _hist(inp, tgt)
    res = _select(cnt)
    return res[0, 0]


# speedup vs baseline: 315.9437x; 1.0026x over previous
import jax
import jax.numpy as jnp
from jax import lax
from jax.experimental import pallas as pl
from jax.experimental.pallas import tpu as pltpu
from jax.experimental.pallas import tpu_sc as plsc

TRIM_FRAC = 0.1
N = 4 * 4096 * 1024
TRIM = int(N * TRIM_FRAC)
KEPT = N - 2 * TRIM

NB = 16384
SHIFT = 17
NW = 32
LANES = 16

NCOL = 1024
NROW = N // NCOL
RPW = NROW // NW
CROWS = 16
NCH = RPW // CROWS
UNROLL = 8

RB, CB = NB // 128, 128


def _hist_body(inp, tgt, cnt_out,
               in_bufs, tg_bufs, cnt_h, in_sems, tg_sems):
    wid = lax.axis_index("s") * 2 + lax.axis_index("c")
    base = wid * RPW

    @plsc.parallel_loop(0, NB // LANES, 1, unroll=UNROLL)
    def _zero(i):
        r = lax.shift_right_logical(i, 3)
        o = (i & 7) * LANES
        cnt_h[r, pl.ds(o, LANES)] = jnp.zeros((LANES,), jnp.int32)

    ones = jnp.ones((LANES,), jnp.int32)
    shift = jnp.full((LANES,), SHIFT, jnp.int32)
    shift_hi = jnp.full((LANES,), SHIFT + 7, jnp.int32)
    colmask = jnp.full((LANES,), CB - 1, jnp.int32)

    def copies(g, b):
        row0 = base + g * CROWS
        return (
            pltpu.make_async_copy(inp.at[pl.ds(row0, CROWS)], in_bufs[b],
                                  in_sems.at[b]),
            pltpu.make_async_copy(tgt.at[pl.ds(row0, CROWS)], tg_bufs[b],
                                  tg_sems.at[b]),
        )

    for b in range(2):
        for c in copies(b, b):
            c.start()

    def chunk_pair(j, acc):
        for b in range(2):
            g = 2 * j + b
            for c in copies(g, b):
                c.wait()

            @plsc.parallel_loop(0, CROWS * NCOL // LANES, 1, unroll=UNROLL,
                                carry=acc)
            def _inner(i, a_sum, b=b):
                r = lax.shift_right_logical(i, 6)
                o = (i & (NCOL // LANES - 1)) * LANES
                a = in_bufs[b][r, pl.ds(o, LANES)]
                t = tg_bufs[b][r, pl.ds(o, LANES)]
                d = a - t
                e = d * d
                bits = lax.bitcast_convert_type(e, jnp.int32)
                brow = lax.shift_right_logical(bits, shift_hi)
                bcol = lax.shift_right_logical(bits, shift) & colmask
                plsc.addupdate_scatter(cnt_h, [brow, bcol], ones)
                return a_sum + e

            acc = _inner

            @pl.when(g + 2 < NCH)
            def _(g=g, b=b):
                for c in copies(g + 2, b):
                    c.start()
        return acc

    acc = lax.fori_loop(0, NCH // 2, chunk_pair,
                        jnp.zeros((LANES,), jnp.float32))

    cnt_h[RB - 1, pl.ds(CB - LANES, LANES)] = (
        lax.bitcast_convert_type(acc, jnp.int32))
    pltpu.sync_copy(cnt_h, cnt_out.at[wid])


_hist = pl.kernel(
    _hist_body,
    out_type=jax.ShapeDtypeStruct((NW, RB, CB), jnp.int32),
    mesh=plsc.VectorSubcoreMesh(core_axis_name="c", subcore_axis_name="s"),
    compiler_params=pltpu.CompilerParams(
        needs_layout_passes=False,
        use_tc_tiling_on_sc=True,
    ),
    scratch_types=[
        [pltpu.VMEM((CROWS, NCOL), jnp.float32)] * 2,
        [pltpu.VMEM((CROWS, NCOL), jnp.float32)] * 2,
        pltpu.VMEM((RB, CB), jnp.int32),
        pltpu.SemaphoreType.DMA((2,)),
        pltpu.SemaphoreType.DMA((2,)),
    ],
)


def _select_body(cnt_ref, out_ref):
    raw = cnt_ref[...]

    rr = lax.broadcasted_iota(jnp.int32, (RB, CB), 0)
    cc = lax.broadcasted_iota(jnp.int32, (RB, CB), 1)
    stash_mask = jnp.logical_and(rr == RB - 1, cc >= CB - LANES)

    total = jnp.sum(jnp.where(stash_mask[None],
                              lax.bitcast_convert_type(raw, jnp.float32),
                              0.0))

    cnt = jnp.sum(raw.astype(jnp.float32), axis=0)
    cnt = jnp.where(stash_mask, 0.0, cnt)

    pat = (rr * CB + cc) * (2 ** SHIFT) + 2 ** (SHIFT - 1)
    mid = lax.bitcast_convert_type(pat, jnp.float32)
    mid = jnp.where(jnp.isfinite(mid), mid, 0.0)
    csum = cnt * mid

    ii = lax.broadcasted_iota(jnp.int32, (CB, CB), 0)
    jj = lax.broadcasted_iota(jnp.int32, (CB, CB), 1)
    upper_inc = (ii <= jj).astype(jnp.float32)
    rowcum = jnp.dot(cnt, upper_inc, preferred_element_type=jnp.float32,
                     precision=lax.Precision.HIGHEST)

    ri = lax.broadcasted_iota(jnp.int32, (RB, RB), 0)
    rj = lax.broadcasted_iota(jnp.int32, (RB, RB), 1)
    lower_strict = (rj < ri).astype(jnp.float32)
    rowtot = jnp.sum(cnt, axis=1, keepdims=True)
    prevrows = jnp.dot(lower_strict, rowtot,
                       preferred_element_type=jnp.float32,
                       precision=lax.Precision.HIGHEST)

    cum = rowcum + prevrows
    cum_prev = cum - cnt

    k1 = jnp.float32(TRIM)
    k2 = jnp.float32(N - TRIM)
    zero = jnp.zeros_like(csum)

    sum_below = jnp.sum(jnp.where(cum <= k1, csum, zero))
    lo_bnd = jnp.logical_and(cum_prev < k1, cum > k1)
    lo_prev = jnp.sum(jnp.where(lo_bnd, cum_prev, zero))
    mid_lo = jnp.sum(jnp.where(lo_bnd, mid, zero))
    bottom = sum_below + (k1 - lo_prev) * mid_lo

    sum_above = jnp.sum(jnp.where(cum_prev >= k2, csum, zero))
    hi_bnd = jnp.logical_and(cum_prev < k2, cum > k2)
    hi_cum = jnp.sum(jnp.where(hi_bnd, cum, zero))
    mid_hi = jnp.sum(jnp.where(hi_bnd, mid, zero))
    top = sum_above + (hi_cum - k2) * mid_hi

    out_ref[0, 0] = (total - bottom - top) / jnp.float32(KEPT)


_select = pl.pallas_call(
    _select_body,
    out_shape=jax.ShapeDtypeStruct((1, 1), jnp.float32),
    out_specs=pl.BlockSpec(memory_space=pltpu.SMEM),
)


@jax.jit
def kernel(input, target):
    inp = input.reshape(NROW, NCOL)
    tgt = target.reshape(NROW, NCOL)
    cnt = _hist(inp, tgt)
    res = _select(cnt)
    return res[0, 0]
